# Initial kernel scaffold; baseline (speedup 1.0000x reference)
import jax
import jax.numpy as jnp
from jax.experimental import pallas as pl


def kernel(x, conv1_w, conv1_b, conv2_w, conv2_b, fc1_w, fc1_b, fc2_w, fc2_b):
    raise NotImplementedError("write your pallas kernel here")


# speedup vs baseline: 7.8626x; 7.8626x over previous
import functools

import jax
import jax.numpy as jnp
from jax import lax
from jax.experimental import pallas as pl
from jax.experimental.pallas import tpu as pltpu

B = 256
N = 256
C = 64
K = 9
CO = 128
BB = 32


def _conv1_body(x_ref, w_ref, b_ref, o_ref, xp_ref):
    xp_ref[...] = jnp.zeros_like(xp_ref)
    xp_ref[:, :, 1:33, 1:33] = (
        x_ref[...].astype(jnp.bfloat16).astype(jnp.float32))
    for co in range(16):
        acc = jnp.zeros((BB, 32, 32), dtype=jnp.float32)
        for ci in range(3):
            for dy in range(3):
                for dx in range(3):
                    w = (w_ref[co, ci, dy, dx]
                         .astype(jnp.bfloat16).astype(jnp.float32))
                    acc = acc + w * xp_ref[:, ci, dy:dy + 32, dx:dx + 32]
        o_ref[:, co, :, :] = jnp.maximum(acc + b_ref[0, co], 0.0)


def _conv1(x, conv1_w, conv1_b):
    return pl.pallas_call(
        _conv1_body,
        grid=(B // BB,),
        in_specs=[
            pl.BlockSpec((BB, 3, 32, 32), lambda i: (i, 0, 0, 0)),
            pl.BlockSpec((16, 3, 3, 3), lambda i: (0, 0, 0, 0)),
            pl.BlockSpec((1, 16), lambda i: (0, 0)),
        ],
        out_specs=pl.BlockSpec((BB, 16, 32, 32), lambda i: (i, 0, 0, 0)),
        out_shape=jax.ShapeDtypeStruct((B, 16, 32, 32), jnp.float32),
        scratch_shapes=[pltpu.VMEM((BB, 3, 34, 34), jnp.float32)],
    )(x, conv1_w, conv1_b)


def _nn_body(t_ref, w_ref, b_ref, o_ref):
    t = t_ref[0]
    ss = jnp.sum(t * t, axis=1, keepdims=True)
    th = (t / (jnp.sqrt(ss) + 1e-12)).astype(jnp.bfloat16)
    sim = lax.dot_general(th, th, (((1,), (1,)), ((), ())),
                          preferred_element_type=jnp.float32)
    col = lax.broadcasted_iota(jnp.int32, (N, N), 1)
    tb = t.astype(jnp.bfloat16)
    acc = jnp.zeros((N, CO), jnp.float32)
    simw = sim
    for k in range(K):
        m = jnp.max(simw, axis=1, keepdims=True)
        eq = simw == m
        idxk = jnp.min(jnp.where(eq, col, N), axis=1, keepdims=True)
        hot = col == idxk
        simw = jnp.where(hot, -jnp.inf, simw)
        p = hot.astype(jnp.bfloat16)
        g = lax.dot_general(p, tb, (((1,), (0,)), ((), ())),
                            preferred_element_type=jnp.float32)
        g = g.astype(jnp.bfloat16)
        wk = w_ref[k].astype(jnp.bfloat16)
        acc = acc + lax.dot_general(g, wk, (((1,), (0,)), ((), ())),
                                    preferred_element_type=jnp.float32)
    out = jnp.maximum(acc + b_ref[...], 0.0)
    o_ref[0] = out.astype(jnp.bfloat16)


def _nn(tokens, conv2_w, conv2_b):
    w = conv2_w.transpose(2, 1, 0)
    return pl.pallas_call(
        _nn_body,
        grid=(B,),
        in_specs=[
            pl.BlockSpec((1, N, C), lambda b: (b, 0, 0)),
            pl.BlockSpec((K, C, CO), lambda b: (0, 0, 0)),
            pl.BlockSpec((1, CO), lambda b: (0, 0)),
        ],
        out_specs=pl.BlockSpec((1, N, CO), lambda b: (b, 0, 0)),
        out_shape=jax.ShapeDtypeStruct((B, N, CO), jnp.bfloat16),
    )(tokens, w, conv2_b.reshape(1, CO))


FCB = 4096


def _fc_body(a_ref, w1_ref, b1_ref, w2_ref, b2_ref, o_ref, acc_ref):
    i = pl.program_id(0)

    @pl.when(i == 0)
    def _():
        acc_ref[...] = jnp.zeros_like(acc_ref)

    a = a_ref[...]
    w1 = w1_ref[...].astype(jnp.bfloat16)
    acc_ref[...] += lax.dot_general(a, w1, (((1,), (1,)), ((), ())),
                                    preferred_element_type=jnp.float32)

    @pl.when(i == pl.num_programs(0) - 1)
    def _():
        z = jnp.maximum(acc_ref[...] + b1_ref[...], 0.0)
        zb = z.astype(jnp.bfloat16)
        w2 = w2_ref[...].astype(jnp.bfloat16)
        o = lax.dot_general(zb, w2, (((1,), (0,)), ((), ())),
                            preferred_element_type=jnp.float32)
        o_ref[...] = o + b2_ref[...]


def _fc(a, fc1_w, fc1_b, fc2_w, fc2_b):
    nk = a.shape[1] // FCB
    return pl.pallas_call(
        _fc_body,
        grid=(nk,),
        in_specs=[
            pl.BlockSpec((B, FCB), lambda i: (0, i)),
            pl.BlockSpec((1024, FCB), lambda i: (0, i)),
            pl.BlockSpec((1, 1024), lambda i: (0, 0)),
            pl.BlockSpec((1024, 10), lambda i: (0, 0)),
            pl.BlockSpec((1, 10), lambda i: (0, 0)),
        ],
        out_specs=pl.BlockSpec((B, 10), lambda i: (0, 0)),
        out_shape=jax.ShapeDtypeStruct((B, 10), jnp.float32),
        scratch_shapes=[pltpu.VMEM((B, 1024), jnp.float32)],
    )(a, fc1_w, fc1_b.reshape(1, 1024), fc2_w.T, fc2_b.reshape(1, 10))


@functools.partial(jax.jit, static_argnums=())
def kernel(x, conv1_w, conv1_b, conv2_w, conv2_b, fc1_w, fc1_b, fc2_w, fc2_b):
    y = _conv1(x, conv1_w, conv1_b.reshape(1, 16))
    tokens = (y.reshape(B, 16, 16, 2, 16, 2)
               .transpose(0, 2, 4, 1, 3, 5)
               .reshape(B, N, C))
    o = _nn(tokens, conv2_w, conv2_b)
    a = (o.reshape(B, 16, 16, 32, 2, 2)
          .transpose(0, 3, 1, 4, 2, 5)
          .reshape(B, 32768))
    return _fc(a, fc1_w, fc1_b, fc2_w, fc2_b)


# speedup vs baseline: 9.2555x; 1.1772x over previous
import functools

import jax
import jax.numpy as jnp
from jax import lax
from jax.experimental import pallas as pl
from jax.experimental.pallas import tpu as pltpu

B = 256
N = 256
C = 64
K = 9
CO = 128
BB = 32


def _conv1_body(x_ref, w_ref, b_ref, o_ref, xp_ref):
    xp_ref[...] = jnp.zeros_like(xp_ref)
    xp_ref[:, :, 1:33, 1:33] = (
        x_ref[...].astype(jnp.bfloat16).astype(jnp.float32))
    for co in range(16):
        acc = jnp.zeros((BB, 32, 32), dtype=jnp.float32)
        for ci in range(3):
            for dy in range(3):
                for dx in range(3):
                    w = (w_ref[co, ci, dy, dx]
                         .astype(jnp.bfloat16).astype(jnp.float32))
                    acc = acc + w * xp_ref[:, ci, dy:dy + 32, dx:dx + 32]
        o_ref[:, co, :, :] = jnp.maximum(acc + b_ref[0, co], 0.0)


def _conv1(x, conv1_w, conv1_b):
    return pl.pallas_call(
        _conv1_body,
        grid=(B // BB,),
        in_specs=[
            pl.BlockSpec((BB, 3, 32, 32), lambda i: (i, 0, 0, 0)),
            pl.BlockSpec((16, 3, 3, 3), lambda i: (0, 0, 0, 0)),
            pl.BlockSpec((1, 16), lambda i: (0, 0)),
        ],
        out_specs=pl.BlockSpec((BB, 16, 32, 32), lambda i: (i, 0, 0, 0)),
        out_shape=jax.ShapeDtypeStruct((B, 16, 32, 32), jnp.float32),
        scratch_shapes=[pltpu.VMEM((BB, 3, 34, 34), jnp.float32)],
        compiler_params=pltpu.CompilerParams(
            dimension_semantics=("parallel",)),
    )(x, conv1_w, conv1_b)


def _nn_body(t_ref, w_ref, b_ref, o_ref):
    t = t_ref[0]
    ss = jnp.sum(t * t, axis=1, keepdims=True)
    th = (t / (jnp.sqrt(ss) + 1e-12)).astype(jnp.bfloat16)
    sim = lax.dot_general(th, th, (((1,), (1,)), ((), ())),
                          preferred_element_type=jnp.float32)
    row = lax.broadcasted_iota(jnp.int32, (N, N), 0)
    tb = t.astype(jnp.bfloat16)
    acc = jnp.zeros((N, CO), jnp.float32)
    simw = sim
    for k in range(K):
        m = jnp.max(simw, axis=0, keepdims=True)
        eq = simw == m
        idxk = jnp.min(jnp.where(eq, row, N), axis=0, keepdims=True)
        hot = row == idxk
        simw = jnp.where(hot, -jnp.inf, simw)
        p = hot.astype(jnp.bfloat16)
        g = lax.dot_general(p, tb, (((0,), (0,)), ((), ())),
                            preferred_element_type=jnp.float32)
        g = g.astype(jnp.bfloat16)
        wk = w_ref[k].astype(jnp.bfloat16)
        acc = acc + lax.dot_general(g, wk, (((1,), (0,)), ((), ())),
                                    preferred_element_type=jnp.float32)
    out = jnp.maximum(acc + b_ref[...], 0.0)
    o_ref[0] = out.astype(jnp.bfloat16)


def _nn(tokens, conv2_w, conv2_b):
    w = conv2_w.transpose(2, 1, 0)
    return pl.pallas_call(
        _nn_body,
        grid=(B,),
        in_specs=[
            pl.BlockSpec((1, N, C), lambda b: (b, 0, 0)),
            pl.BlockSpec((K, C, CO), lambda b: (0, 0, 0)),
            pl.BlockSpec((1, CO), lambda b: (0, 0)),
        ],
        out_specs=pl.BlockSpec((1, N, CO), lambda b: (b, 0, 0)),
        out_shape=jax.ShapeDtypeStruct((B, N, CO), jnp.bfloat16),
        compiler_params=pltpu.CompilerParams(
            dimension_semantics=("parallel",)),
    )(tokens, w, conv2_b.reshape(1, CO))


FCB = 4096


def _fc_body(a_ref, w1_ref, b1_ref, w2_ref, b2_ref, o_ref, acc_ref):
    i = pl.program_id(0)

    @pl.when(i == 0)
    def _():
        acc_ref[...] = jnp.zeros_like(acc_ref)

    a = a_ref[...]
    w1 = w1_ref[...].astype(jnp.bfloat16)
    acc_ref[...] += lax.dot_general(a, w1, (((1,), (1,)), ((), ())),
                                    preferred_element_type=jnp.float32)

    @pl.when(i == pl.num_programs(0) - 1)
    def _():
        z = jnp.maximum(acc_ref[...] + b1_ref[...], 0.0)
        zb = z.astype(jnp.bfloat16)
        w2 = w2_ref[...].astype(jnp.bfloat16)
        o = lax.dot_general(zb, w2, (((1,), (0,)), ((), ())),
                            preferred_element_type=jnp.float32)
        o_ref[...] = o + b2_ref[...]


def _fc(a, fc1_w, fc1_b, fc2_w, fc2_b):
    nk = a.shape[1] // FCB
    return pl.pallas_call(
        _fc_body,
        grid=(nk,),
        in_specs=[
            pl.BlockSpec((B, FCB), lambda i: (0, i)),
            pl.BlockSpec((1024, FCB), lambda i: (0, i)),
            pl.BlockSpec((1, 1024), lambda i: (0, 0)),
            pl.BlockSpec((1024, 10), lambda i: (0, 0)),
            pl.BlockSpec((1, 10), lambda i: (0, 0)),
        ],
        out_specs=pl.BlockSpec((B, 10), lambda i: (0, 0)),
        out_shape=jax.ShapeDtypeStruct((B, 10), jnp.float32),
        scratch_shapes=[pltpu.VMEM((B, 1024), jnp.float32)],
    )(a, fc1_w, fc1_b.reshape(1, 1024), fc2_w.T, fc2_b.reshape(1, 10))


@functools.partial(jax.jit, static_argnums=())
def kernel(x, conv1_w, conv1_b, conv2_w, conv2_b, fc1_w, fc1_b, fc2_w, fc2_b):
    y = _conv1(x, conv1_w, conv1_b.reshape(1, 16))
    tokens = (y.reshape(B, 16, 16, 2, 16, 2)
               .transpose(0, 2, 4, 1, 3, 5)
               .reshape(B, N, C))
    o = _nn(tokens, conv2_w, conv2_b)
    a = (o.reshape(B, 16, 16, 32, 2, 2)
          .transpose(0, 3, 1, 4, 2, 5)
          .reshape(B, 32768))
    return _fc(a, fc1_w, fc1_b, fc2_w, fc2_b)
